# baseline (speedup 1.0000x reference)
import functools

import jax
import jax.numpy as jnp
from jax import lax
from jax.experimental import pallas as pl
from jax.experimental.pallas import tpu as pltpu
from jax.experimental.pallas import tpu_sc as plsc

_N = 10000
_D = 128
_E = 320000

_NC = 2
_NS = 16
_NW = _NC * _NS
_EPW = _E // _NW
_CHUNK = 80
_NCHUNK = 128
_EPWP = _NCHUNK * _CHUNK
_NP = 10240
_RPT = _NP // _NS
_PADROW = _NP - _CHUNK

_mesh = plsc.VectorSubcoreMesh(core_axis_name="c", subcore_axis_name="s")


@functools.partial(
    pl.kernel,
    out_type=jax.ShapeDtypeStruct((_NC, _NP, _D), jnp.float32),
    mesh=_mesh,
    scratch_types=[
        pltpu.VMEM((_CHUNK,), jnp.int32),
        pltpu.VMEM((_CHUNK,), jnp.int32),
        pltpu.VMEM((_CHUNK,), jnp.int32),
        pltpu.VMEM((_CHUNK,), jnp.int32),
        pltpu.VMEM((_CHUNK, _D), jnp.float32),
        pltpu.VMEM((_CHUNK, _D), jnp.float32),
        pltpu.VMEM_SHARED((_NP, _D), jnp.float32),
        pltpu.SemaphoreType.DMA,
        pltpu.SemaphoreType.DMA,
        pltpu.SemaphoreType.DMA,
        pltpu.SemaphoreType.DMA,
    ])
def _sc_segsum(h_hbm, src_hbm, dst_hbm, z_hbm, sum_out,
               srcA, srcB, dstA, dstB, rows0, rows1, acc_sh,
               semIA, semIB, sem0, sem1):
    c = lax.axis_index("c")
    s = lax.axis_index("s")
    wid = c * _NS + s

    r0 = s * _RPT
    pltpu.sync_copy(z_hbm.at[pl.ds(r0, _RPT)], acc_sh.at[pl.ds(r0, _RPT)])
    pltpu.sync_copy(src_hbm.at[wid, 0], srcA)
    pltpu.sync_copy(dst_hbm.at[wid, 0], dstA)
    plsc.subcore_barrier()

    def body(j, carry):
        i = 2 * j
        dg0 = pltpu.async_copy(h_hbm.at[srcA], rows0, sem0)
        pltpu.sync_copy(src_hbm.at[wid, i + 1], srcB)
        pltpu.sync_copy(dst_hbm.at[wid, i + 1], dstB)
        dg0.wait()
        dg1 = pltpu.async_copy(h_hbm.at[srcB], rows1, sem1)
        pltpu.sync_copy(rows0, acc_sh.at[dstA], add=True)
        pltpu.sync_copy(src_hbm.at[wid, i + 2], srcA)
        pltpu.sync_copy(dst_hbm.at[wid, i + 2], dstA)
        dg1.wait()
        pltpu.sync_copy(rows1, acc_sh.at[dstB], add=True)
        return carry

    lax.fori_loop(0, _NCHUNK // 2, body, 0)
    plsc.subcore_barrier()

    pltpu.sync_copy(acc_sh.at[pl.ds(r0, _RPT)],
                    sum_out.at[c, pl.ds(r0, _RPT)])


@functools.partial(
    pl.kernel,
    out_type=jax.ShapeDtypeStruct((_NC, _NP, _D), jnp.float32),
    mesh=_mesh,
    scratch_types=[
        pltpu.VMEM((_NCHUNK, _CHUNK), jnp.int32),
        pltpu.VMEM((_CHUNK, _D), jnp.float32),
        pltpu.VMEM_SHARED((_NP, _D), jnp.float32),
    ])
def _sc_degree(dst_hbm, z_hbm, ones_hbm, cnt_out, dst_v, ones_v, cnt_sh):
    c = lax.axis_index("c")
    s = lax.axis_index("s")

    pltpu.sync_copy(dst_hbm.at[c * _NS + s], dst_v)
    r0 = s * _RPT
    pltpu.sync_copy(z_hbm.at[pl.ds(r0, _RPT)], cnt_sh.at[pl.ds(r0, _RPT)])
    pltpu.sync_copy(ones_hbm, ones_v)
    plsc.subcore_barrier()

    def body(i, carry):
        pltpu.sync_copy(ones_v, cnt_sh.at[dst_v.at[i]], add=True)
        return carry

    lax.fori_loop(0, _NCHUNK, body, 0)
    plsc.subcore_barrier()

    pltpu.sync_copy(cnt_sh.at[pl.ds(r0, _RPT)],
                    cnt_out.at[c, pl.ds(r0, _RPT)])


_BR = 400


def _tc_layer(h, sum2, cnt2, WlT, WrT, bl2d):
    def body(h_ref, s_ref, c_ref, wl_ref, wr_ref, b_ref, o_ref):
        ssum = s_ref[0] + s_ref[1]
        cnt = c_ref[0][:, 0:1] + c_ref[1][:, 0:1]
        mean = ssum / jnp.maximum(cnt, 1.0)
        acc = jax.lax.dot(mean, wl_ref[...],
                          precision=jax.lax.Precision.HIGHEST,
                          preferred_element_type=jnp.float32)
        acc = acc + jax.lax.dot(h_ref[...], wr_ref[...],
                                precision=jax.lax.Precision.HIGHEST,
                                preferred_element_type=jnp.float32)
        o_ref[...] = jnp.maximum(acc + b_ref[...], 0.0)

    return pl.pallas_call(
        body,
        grid=(_N // _BR,),
        in_specs=[
            pl.BlockSpec((_BR, _D), lambda i: (i, 0)),
            pl.BlockSpec((_NC, _BR, _D), lambda i: (0, i, 0)),
            pl.BlockSpec((_NC, _BR, _D), lambda i: (0, i, 0)),
            pl.BlockSpec((_D, _D), lambda i: (0, 0)),
            pl.BlockSpec((_D, _D), lambda i: (0, 0)),
            pl.BlockSpec((1, _D), lambda i: (0, 0)),
        ],
        out_specs=pl.BlockSpec((_BR, _D), lambda i: (i, 0)),
        out_shape=jax.ShapeDtypeStruct((_N, _D), jnp.float32),
    )(h, sum2, cnt2, WlT, WrT, bl2d)


def kernel(x, edge_index, Wl1, bl1, Wr1, Wl2, bl2, Wr2, Wl3, bl3, Wr3):
    pad = _EPWP - _EPW
    srcw = edge_index[0].reshape(_NW, _EPW)
    dstw = edge_index[1].reshape(_NW, _EPW)
    src3 = jnp.pad(srcw, ((0, 0), (0, pad))).reshape(_NW, _NCHUNK, _CHUNK)
    dst3 = jnp.pad(dstw, ((0, 0), (0, pad)),
                   constant_values=_PADROW).reshape(_NW, _NCHUNK, _CHUNK)
    src3p = jnp.pad(src3, ((0, 0), (0, 2), (0, 0)))
    dst3p = jnp.pad(dst3, ((0, 0), (0, 2), (0, 0)),
                    constant_values=_PADROW)
    zeros = jnp.zeros((_NP, _D), jnp.float32)
    ones = jnp.ones((_CHUNK, _D), jnp.float32)

    h = x
    cnt2 = _sc_degree(dst3, zeros, ones)
    for Wl, bl, Wr in [(Wl1, bl1, Wr1), (Wl2, bl2, Wr2), (Wl3, bl3, Wr3)]:
        sum2 = _sc_segsum(h, src3p, dst3p, zeros)
        h = _tc_layer(h, sum2[:, :_N], cnt2[:, :_N], Wl.T, Wr.T,
                      bl.reshape(1, _D))
    return h


# speedup vs baseline: 1.4622x; 1.4622x over previous
import functools

import jax
import jax.numpy as jnp
from jax import lax
from jax.experimental import pallas as pl
from jax.experimental.pallas import tpu as pltpu
from jax.experimental.pallas import tpu_sc as plsc

_N = 10000
_D = 128
_E = 320000

_NC = 2
_NS = 16
_NW = _NC * _NS
_EPW = _E // _NW
_CHUNK = 80
_NCHUNK = 128
_EPWP = _NCHUNK * _CHUNK
_NP = 10240
_RPT = _NP // _NS
_PADROW = _NP - _CHUNK

_mesh = plsc.VectorSubcoreMesh(core_axis_name="c", subcore_axis_name="s")


_SCH = 80
_SNCH = _EPW // _SCH


@functools.partial(
    pl.kernel,
    out_type=jax.ShapeDtypeStruct((_NC, _NP, _D), jnp.float32),
    mesh=_mesh,
    scratch_types=[
        pltpu.VMEM((_SCH,), jnp.int32),
        pltpu.VMEM((_SCH,), jnp.int32),
        pltpu.VMEM((_SCH, _D), jnp.float32),
        pltpu.VMEM_SHARED((_NP, _D), jnp.float32),
        pltpu.SemaphoreType.DMA,
    ])
def _sc_segsum(h_hbm, src_hbm, dst_hbm, z_hbm, sum_out,
               src_v, dst_v, rows_v, acc_sh, sem):
    c = lax.axis_index("c")
    s = lax.axis_index("s")
    wid = c * _NS + s

    r0 = s * _RPT
    pltpu.sync_copy(z_hbm.at[pl.ds(r0, _RPT)], acc_sh.at[pl.ds(r0, _RPT)])
    plsc.subcore_barrier()

    def body(i, carry):
        base = pl.multiple_of(wid * _EPW + i * _SCH, 8)
        pltpu.sync_copy(src_hbm.at[pl.ds(base, _SCH)], src_v)
        pltpu.sync_copy(dst_hbm.at[pl.ds(base, _SCH)], dst_v)
        pltpu.async_copy(h_hbm.at[src_v], rows_v, sem).wait()
        pltpu.sync_copy(rows_v, acc_sh.at[dst_v], add=True)
        return carry

    lax.fori_loop(0, _SNCH, body, 0)
    plsc.subcore_barrier()

    pltpu.sync_copy(acc_sh.at[pl.ds(r0, _RPT)],
                    sum_out.at[c, pl.ds(r0, _RPT)])


@functools.partial(
    pl.kernel,
    out_type=jax.ShapeDtypeStruct((_NC, _NP, _D), jnp.float32),
    mesh=_mesh,
    scratch_types=[
        pltpu.VMEM((_NCHUNK, _CHUNK), jnp.int32),
        pltpu.VMEM((_CHUNK, _D), jnp.float32),
        pltpu.VMEM_SHARED((_NP, _D), jnp.float32),
    ])
def _sc_degree(dst_hbm, z_hbm, ones_hbm, cnt_out, dst_v, ones_v, cnt_sh):
    c = lax.axis_index("c")
    s = lax.axis_index("s")

    pltpu.sync_copy(dst_hbm.at[c * _NS + s], dst_v)
    r0 = s * _RPT
    pltpu.sync_copy(z_hbm.at[pl.ds(r0, _RPT)], cnt_sh.at[pl.ds(r0, _RPT)])
    pltpu.sync_copy(ones_hbm, ones_v)
    plsc.subcore_barrier()

    def body(i, carry):
        pltpu.sync_copy(ones_v, cnt_sh.at[dst_v.at[i]], add=True)
        return carry

    lax.fori_loop(0, _NCHUNK, body, 0)
    plsc.subcore_barrier()

    pltpu.sync_copy(cnt_sh.at[pl.ds(r0, _RPT)],
                    cnt_out.at[c, pl.ds(r0, _RPT)])


_BR = 400


def _tc_layer(h, sum2, cnt2, WlT, WrT, bl2d):
    def body(h_ref, s_ref, c_ref, wl_ref, wr_ref, b_ref, o_ref):
        ssum = s_ref[0] + s_ref[1]
        cnt = c_ref[0][:, 0:1] + c_ref[1][:, 0:1]
        mean = ssum / jnp.maximum(cnt, 1.0)
        acc = jax.lax.dot(mean, wl_ref[...],
                          precision=jax.lax.Precision.HIGHEST,
                          preferred_element_type=jnp.float32)
        acc = acc + jax.lax.dot(h_ref[...], wr_ref[...],
                                precision=jax.lax.Precision.HIGHEST,
                                preferred_element_type=jnp.float32)
        o_ref[...] = jnp.maximum(acc + b_ref[...], 0.0)

    return pl.pallas_call(
        body,
        grid=(_N // _BR,),
        in_specs=[
            pl.BlockSpec((_BR, _D), lambda i: (i, 0)),
            pl.BlockSpec((_NC, _BR, _D), lambda i: (0, i, 0)),
            pl.BlockSpec((_NC, _BR, _D), lambda i: (0, i, 0)),
            pl.BlockSpec((_D, _D), lambda i: (0, 0)),
            pl.BlockSpec((_D, _D), lambda i: (0, 0)),
            pl.BlockSpec((1, _D), lambda i: (0, 0)),
        ],
        out_specs=pl.BlockSpec((_BR, _D), lambda i: (i, 0)),
        out_shape=jax.ShapeDtypeStruct((_N, _D), jnp.float32),
    )(h, sum2, cnt2, WlT, WrT, bl2d)


def kernel(x, edge_index, Wl1, bl1, Wr1, Wl2, bl2, Wr2, Wl3, bl3, Wr3):
    pad = _EPWP - _EPW
    srcw = edge_index[0].reshape(_NW, _EPW)
    dstw = edge_index[1].reshape(_NW, _EPW)
    src3 = jnp.pad(srcw, ((0, 0), (0, pad))).reshape(_NW, _NCHUNK, _CHUNK)
    dst3 = jnp.pad(dstw, ((0, 0), (0, pad)),
                   constant_values=_PADROW).reshape(_NW, _NCHUNK, _CHUNK)
    zeros = jnp.zeros((_NP, _D), jnp.float32)
    ones = jnp.ones((_CHUNK, _D), jnp.float32)

    h = x
    cnt2 = _sc_degree(dst3, zeros, ones)
    for Wl, bl, Wr in [(Wl1, bl1, Wr1), (Wl2, bl2, Wr2), (Wl3, bl3, Wr3)]:
        sum2 = _sc_segsum(h, edge_index[0], edge_index[1], zeros)
        h = _tc_layer(h, sum2[:, :_N], cnt2[:, :_N], Wl.T, Wr.T,
                      bl.reshape(1, _D))
    return h


# speedup vs baseline: 2.0082x; 1.3735x over previous
import functools

import jax
import jax.numpy as jnp
from jax import lax
from jax.experimental import pallas as pl
from jax.experimental.pallas import tpu as pltpu
from jax.experimental.pallas import tpu_sc as plsc

_N = 10000
_D = 128
_E = 320000

_NC = 2
_NS = 16
_NW = _NC * _NS
_EPW = _E // _NW
_CHUNK = 80
_NCHUNK = 128
_EPWP = _NCHUNK * _CHUNK
_NP = 10240
_RPT = _NP // _NS
_PADROW = _NP - _CHUNK

_mesh = plsc.VectorSubcoreMesh(core_axis_name="c", subcore_axis_name="s")


_SCH = 80
_SNCH = _EPW // _SCH


@functools.partial(
    pl.kernel,
    out_type=jax.ShapeDtypeStruct((_NC, _NP, _D), jnp.float32),
    mesh=_mesh,
    scratch_types=[
        pltpu.VMEM((_SCH,), jnp.int32),
        pltpu.VMEM((_SCH,), jnp.int32),
        pltpu.VMEM((_SCH,), jnp.int32),
        pltpu.VMEM((_SCH,), jnp.int32),
        pltpu.VMEM((_SCH, _D), jnp.float32),
        pltpu.VMEM_SHARED((_NP, _D), jnp.float32),
        pltpu.SemaphoreType.DMA,
    ])
def _sc_segsum(h_hbm, src_hbm, dst_hbm, z_hbm, sum_out,
               srcA, dstA, srcB, dstB, rows_v, acc_sh, sem):
    c = lax.axis_index("c")
    s = lax.axis_index("s")
    wid = c * _NS + s

    def load_idx(i, src_v, dst_v):
        base = pl.multiple_of(wid * _EPW + i * _SCH, 8)
        pltpu.sync_copy(src_hbm.at[pl.ds(base, _SCH)], src_v)
        pltpu.sync_copy(dst_hbm.at[pl.ds(base, _SCH)], dst_v)

    r0 = s * _RPT
    pltpu.sync_copy(z_hbm.at[pl.ds(r0, _RPT)], acc_sh.at[pl.ds(r0, _RPT)])
    load_idx(0, srcA, dstA)
    plsc.subcore_barrier()

    def body(j, carry):
        i = 2 * j
        dg = pltpu.async_copy(h_hbm.at[srcA], rows_v, sem)
        load_idx(i + 1, srcB, dstB)
        dg.wait()
        pltpu.sync_copy(rows_v, acc_sh.at[dstA], add=True)
        dg = pltpu.async_copy(h_hbm.at[srcB], rows_v, sem)
        load_idx(i + 2, srcA, dstA)
        dg.wait()
        pltpu.sync_copy(rows_v, acc_sh.at[dstB], add=True)
        return carry

    lax.fori_loop(0, _SNCH // 2, body, 0)
    pltpu.async_copy(h_hbm.at[srcA], rows_v, sem).wait()
    pltpu.sync_copy(rows_v, acc_sh.at[dstA], add=True)
    plsc.subcore_barrier()

    pltpu.sync_copy(acc_sh.at[pl.ds(r0, _RPT)],
                    sum_out.at[c, pl.ds(r0, _RPT)])


@functools.partial(
    pl.kernel,
    out_type=jax.ShapeDtypeStruct((_NC, _NP, _D), jnp.float32),
    mesh=_mesh,
    scratch_types=[
        pltpu.VMEM((_NCHUNK, _CHUNK), jnp.int32),
        pltpu.VMEM((_CHUNK, _D), jnp.float32),
        pltpu.VMEM_SHARED((_NP, _D), jnp.float32),
    ])
def _sc_degree(dst_hbm, z_hbm, ones_hbm, cnt_out, dst_v, ones_v, cnt_sh):
    c = lax.axis_index("c")
    s = lax.axis_index("s")

    pltpu.sync_copy(dst_hbm.at[c * _NS + s], dst_v)
    r0 = s * _RPT
    pltpu.sync_copy(z_hbm.at[pl.ds(r0, _RPT)], cnt_sh.at[pl.ds(r0, _RPT)])
    pltpu.sync_copy(ones_hbm, ones_v)
    plsc.subcore_barrier()

    def body(i, carry):
        pltpu.sync_copy(ones_v, cnt_sh.at[dst_v.at[i]], add=True)
        return carry

    lax.fori_loop(0, _NCHUNK, body, 0)
    plsc.subcore_barrier()

    pltpu.sync_copy(cnt_sh.at[pl.ds(r0, _RPT)],
                    cnt_out.at[c, pl.ds(r0, _RPT)])


_BR = 400


def _tc_layer(h, sum2, cnt2, WlT, WrT, bl2d):
    def body(h_ref, s_ref, c_ref, wl_ref, wr_ref, b_ref, o_ref):
        ssum = s_ref[0] + s_ref[1]
        cnt = c_ref[0][:, 0:1] + c_ref[1][:, 0:1]
        mean = ssum / jnp.maximum(cnt, 1.0)
        acc = jax.lax.dot(mean, wl_ref[...],
                          precision=jax.lax.Precision.HIGHEST,
                          preferred_element_type=jnp.float32)
        acc = acc + jax.lax.dot(h_ref[...], wr_ref[...],
                                precision=jax.lax.Precision.HIGHEST,
                                preferred_element_type=jnp.float32)
        o_ref[...] = jnp.maximum(acc + b_ref[...], 0.0)

    return pl.pallas_call(
        body,
        grid=(_N // _BR,),
        in_specs=[
            pl.BlockSpec((_BR, _D), lambda i: (i, 0)),
            pl.BlockSpec((_NC, _BR, _D), lambda i: (0, i, 0)),
            pl.BlockSpec((_NC, _BR, _D), lambda i: (0, i, 0)),
            pl.BlockSpec((_D, _D), lambda i: (0, 0)),
            pl.BlockSpec((_D, _D), lambda i: (0, 0)),
            pl.BlockSpec((1, _D), lambda i: (0, 0)),
        ],
        out_specs=pl.BlockSpec((_BR, _D), lambda i: (i, 0)),
        out_shape=jax.ShapeDtypeStruct((_N, _D), jnp.float32),
    )(h, sum2, cnt2, WlT, WrT, bl2d)


def kernel(x, edge_index, Wl1, bl1, Wr1, Wl2, bl2, Wr2, Wl3, bl3, Wr3):
    pad = _EPWP - _EPW
    srcw = edge_index[0].reshape(_NW, _EPW)
    dstw = edge_index[1].reshape(_NW, _EPW)
    src3 = jnp.pad(srcw, ((0, 0), (0, pad))).reshape(_NW, _NCHUNK, _CHUNK)
    dst3 = jnp.pad(dstw, ((0, 0), (0, pad)),
                   constant_values=_PADROW).reshape(_NW, _NCHUNK, _CHUNK)
    zeros = jnp.zeros((_NP, _D), jnp.float32)
    ones = jnp.ones((_CHUNK, _D), jnp.float32)

    h = x
    cnt2 = _sc_degree(dst3, zeros, ones)
    for Wl, bl, Wr in [(Wl1, bl1, Wr1), (Wl2, bl2, Wr2), (Wl3, bl3, Wr3)]:
        sum2 = _sc_segsum(h, edge_index[0], edge_index[1], zeros)
        h = _tc_layer(h, sum2[:, :_N], cnt2[:, :_N], Wl.T, Wr.T,
                      bl.reshape(1, _D))
    return h


# speedup vs baseline: 2.1086x; 1.0500x over previous
import functools

import jax
import jax.numpy as jnp
from jax import lax
from jax.experimental import pallas as pl
from jax.experimental.pallas import tpu as pltpu
from jax.experimental.pallas import tpu_sc as plsc

_N = 10000
_D = 128
_E = 320000

_NC = 2
_NS = 16
_NW = _NC * _NS
_EPW = _E // _NW
_CHUNK = 80
_NCHUNK = 128
_EPWP = _NCHUNK * _CHUNK
_NP = 10240
_RPT = _NP // _NS
_PADROW = _NP - _CHUNK

_mesh = plsc.VectorSubcoreMesh(core_axis_name="c", subcore_axis_name="s")


_SCH = 80
_SNCH = _EPW // _SCH


@functools.partial(
    pl.kernel,
    out_type=jax.ShapeDtypeStruct((_NC, _NP, _D), jnp.float32),
    mesh=_mesh,
    scratch_types=[
        pltpu.VMEM((_SCH,), jnp.int32),
        pltpu.VMEM((_SCH,), jnp.int32),
        pltpu.VMEM((_SCH,), jnp.int32),
        pltpu.VMEM((_SCH,), jnp.int32),
        pltpu.VMEM((_SCH, _D), jnp.float32),
        pltpu.VMEM((_SCH, _D), jnp.float32),
        pltpu.VMEM_SHARED((_NP, _D), jnp.float32),
        pltpu.SemaphoreType.DMA,
        pltpu.SemaphoreType.DMA,
    ])
def _sc_segsum(h_hbm, src_hbm, dst_hbm, z_hbm, sum_out,
               srcA, dstA, srcB, dstB, rows0, rows1, acc_sh, sem0, sem1):
    c = lax.axis_index("c")
    s = lax.axis_index("s")
    wid = c * _NS + s

    def load_idx(i, src_v, dst_v):
        base = pl.multiple_of(wid * _EPW + i * _SCH, 8)
        pltpu.sync_copy(src_hbm.at[pl.ds(base, _SCH)], src_v)
        pltpu.sync_copy(dst_hbm.at[pl.ds(base, _SCH)], dst_v)

    r0 = s * _RPT
    pltpu.sync_copy(z_hbm.at[pl.ds(r0, _RPT)], acc_sh.at[pl.ds(r0, _RPT)])
    load_idx(0, srcA, dstA)
    plsc.subcore_barrier()

    def body(j, carry):
        i = 2 * j
        dg0 = pltpu.async_copy(h_hbm.at[srcA], rows0, sem0)
        load_idx(i + 1, srcB, dstB)
        dg1 = pltpu.async_copy(h_hbm.at[srcB], rows1, sem1)
        dg0.wait()
        pltpu.sync_copy(rows0, acc_sh.at[dstA], add=True)
        load_idx(i + 2, srcA, dstA)
        dg1.wait()
        pltpu.sync_copy(rows1, acc_sh.at[dstB], add=True)
        return carry

    lax.fori_loop(0, _SNCH // 2, body, 0)
    pltpu.async_copy(h_hbm.at[srcA], rows0, sem0).wait()
    pltpu.sync_copy(rows0, acc_sh.at[dstA], add=True)
    plsc.subcore_barrier()

    pltpu.sync_copy(acc_sh.at[pl.ds(r0, _RPT)],
                    sum_out.at[c, pl.ds(r0, _RPT)])


@functools.partial(
    pl.kernel,
    out_type=jax.ShapeDtypeStruct((_NC, _NP, _D), jnp.float32),
    mesh=_mesh,
    scratch_types=[
        pltpu.VMEM((_NCHUNK, _CHUNK), jnp.int32),
        pltpu.VMEM((_CHUNK, _D), jnp.float32),
        pltpu.VMEM_SHARED((_NP, _D), jnp.float32),
    ])
def _sc_degree(dst_hbm, z_hbm, ones_hbm, cnt_out, dst_v, ones_v, cnt_sh):
    c = lax.axis_index("c")
    s = lax.axis_index("s")

    pltpu.sync_copy(dst_hbm.at[c * _NS + s], dst_v)
    r0 = s * _RPT
    pltpu.sync_copy(z_hbm.at[pl.ds(r0, _RPT)], cnt_sh.at[pl.ds(r0, _RPT)])
    pltpu.sync_copy(ones_hbm, ones_v)
    plsc.subcore_barrier()

    def body(i, carry):
        pltpu.sync_copy(ones_v, cnt_sh.at[dst_v.at[i]], add=True)
        return carry

    lax.fori_loop(0, _NCHUNK, body, 0)
    plsc.subcore_barrier()

    pltpu.sync_copy(cnt_sh.at[pl.ds(r0, _RPT)],
                    cnt_out.at[c, pl.ds(r0, _RPT)])


_BR = 400


def _tc_layer(h, sum2, cnt2, WlT, WrT, bl2d):
    def body(h_ref, s_ref, c_ref, wl_ref, wr_ref, b_ref, o_ref):
        ssum = s_ref[0] + s_ref[1]
        cnt = c_ref[0][:, 0:1] + c_ref[1][:, 0:1]
        mean = ssum / jnp.maximum(cnt, 1.0)
        acc = jax.lax.dot(mean, wl_ref[...],
                          precision=jax.lax.Precision.HIGHEST,
                          preferred_element_type=jnp.float32)
        acc = acc + jax.lax.dot(h_ref[...], wr_ref[...],
                                precision=jax.lax.Precision.HIGHEST,
                                preferred_element_type=jnp.float32)
        o_ref[...] = jnp.maximum(acc + b_ref[...], 0.0)

    return pl.pallas_call(
        body,
        grid=(_N // _BR,),
        in_specs=[
            pl.BlockSpec((_BR, _D), lambda i: (i, 0)),
            pl.BlockSpec((_NC, _BR, _D), lambda i: (0, i, 0)),
            pl.BlockSpec((_NC, _BR, _D), lambda i: (0, i, 0)),
            pl.BlockSpec((_D, _D), lambda i: (0, 0)),
            pl.BlockSpec((_D, _D), lambda i: (0, 0)),
            pl.BlockSpec((1, _D), lambda i: (0, 0)),
        ],
        out_specs=pl.BlockSpec((_BR, _D), lambda i: (i, 0)),
        out_shape=jax.ShapeDtypeStruct((_N, _D), jnp.float32),
    )(h, sum2, cnt2, WlT, WrT, bl2d)


def kernel(x, edge_index, Wl1, bl1, Wr1, Wl2, bl2, Wr2, Wl3, bl3, Wr3):
    pad = _EPWP - _EPW
    srcw = edge_index[0].reshape(_NW, _EPW)
    dstw = edge_index[1].reshape(_NW, _EPW)
    src3 = jnp.pad(srcw, ((0, 0), (0, pad))).reshape(_NW, _NCHUNK, _CHUNK)
    dst3 = jnp.pad(dstw, ((0, 0), (0, pad)),
                   constant_values=_PADROW).reshape(_NW, _NCHUNK, _CHUNK)
    zeros = jnp.zeros((_NP, _D), jnp.float32)
    ones = jnp.ones((_CHUNK, _D), jnp.float32)

    h = x
    cnt2 = _sc_degree(dst3, zeros, ones)
    for Wl, bl, Wr in [(Wl1, bl1, Wr1), (Wl2, bl2, Wr2), (Wl3, bl3, Wr3)]:
        sum2 = _sc_segsum(h, edge_index[0], edge_index[1], zeros)
        h = _tc_layer(h, sum2[:, :_N], cnt2[:, :_N], Wl.T, Wr.T,
                      bl.reshape(1, _D))
    return h


# speedup vs baseline: 2.1777x; 1.0328x over previous
import functools

import jax
import jax.numpy as jnp
from jax import lax
from jax.experimental import pallas as pl
from jax.experimental.pallas import tpu as pltpu
from jax.experimental.pallas import tpu_sc as plsc

_N = 10000
_D = 128
_E = 320000

_NC = 2
_NS = 16
_NW = _NC * _NS
_EPW = _E // _NW
_CHUNK = 80
_NCHUNK = 128
_EPWP = _NCHUNK * _CHUNK
_NP = 10240
_RPT = _NP // _NS
_PADROW = _NP - _CHUNK

_mesh = plsc.VectorSubcoreMesh(core_axis_name="c", subcore_axis_name="s")


_SCH = 80
_SNCH = _EPW // _SCH


@functools.partial(
    pl.kernel,
    out_type=jax.ShapeDtypeStruct((_NC, _NP, _D), jnp.float32),
    mesh=_mesh,
    scratch_types=(
        [pltpu.VMEM((_SCH,), jnp.int32)] * 8
        + [pltpu.VMEM((_SCH, _D), jnp.float32)] * 4
        + [
            pltpu.VMEM_SHARED((_NP, _D), jnp.float32),
            pltpu.SemaphoreType.DMA,
            pltpu.SemaphoreType.DMA,
            pltpu.SemaphoreType.DMA,
            pltpu.SemaphoreType.DMA,
        ]
    ))
def _sc_segsum(h_hbm, src_hbm, dst_hbm, z_hbm, sum_out,
               srcA, dstA, srcB, dstB, srcC, dstC, srcD, dstD,
               rows0, rows1, rows2, rows3, acc_sh, sem0, sem1, sem2, sem3):
    c = lax.axis_index("c")
    s = lax.axis_index("s")
    wid = c * _NS + s

    def load_idx(i, src_v, dst_v):
        base = pl.multiple_of(wid * _EPW + i * _SCH, 8)
        pltpu.sync_copy(src_hbm.at[pl.ds(base, _SCH)], src_v)
        pltpu.sync_copy(dst_hbm.at[pl.ds(base, _SCH)], dst_v)

    r0 = s * _RPT
    pltpu.sync_copy(z_hbm.at[pl.ds(r0, _RPT)], acc_sh.at[pl.ds(r0, _RPT)])
    load_idx(0, srcA, dstA)
    plsc.subcore_barrier()

    def body(j, carry):
        i = 4 * j
        dg0 = pltpu.async_copy(h_hbm.at[srcA], rows0, sem0)
        load_idx(i + 1, srcB, dstB)
        dg1 = pltpu.async_copy(h_hbm.at[srcB], rows1, sem1)
        load_idx(i + 2, srcC, dstC)
        dg2 = pltpu.async_copy(h_hbm.at[srcC], rows2, sem2)
        load_idx(i + 3, srcD, dstD)
        dg3 = pltpu.async_copy(h_hbm.at[srcD], rows3, sem3)
        dg0.wait()
        pltpu.sync_copy(rows0, acc_sh.at[dstA], add=True)
        load_idx(i + 4, srcA, dstA)
        dg1.wait()
        pltpu.sync_copy(rows1, acc_sh.at[dstB], add=True)
        dg2.wait()
        pltpu.sync_copy(rows2, acc_sh.at[dstC], add=True)
        dg3.wait()
        pltpu.sync_copy(rows3, acc_sh.at[dstD], add=True)
        return carry

    lax.fori_loop(0, _SNCH // 4, body, 0)
    pltpu.async_copy(h_hbm.at[srcA], rows0, sem0).wait()
    pltpu.sync_copy(rows0, acc_sh.at[dstA], add=True)
    plsc.subcore_barrier()

    pltpu.sync_copy(acc_sh.at[pl.ds(r0, _RPT)],
                    sum_out.at[c, pl.ds(r0, _RPT)])


@functools.partial(
    pl.kernel,
    out_type=jax.ShapeDtypeStruct((_NC, _NP, _D), jnp.float32),
    mesh=_mesh,
    scratch_types=[
        pltpu.VMEM((_NCHUNK, _CHUNK), jnp.int32),
        pltpu.VMEM((_CHUNK, _D), jnp.float32),
        pltpu.VMEM_SHARED((_NP, _D), jnp.float32),
    ])
def _sc_degree(dst_hbm, z_hbm, ones_hbm, cnt_out, dst_v, ones_v, cnt_sh):
    c = lax.axis_index("c")
    s = lax.axis_index("s")

    pltpu.sync_copy(dst_hbm.at[c * _NS + s], dst_v)
    r0 = s * _RPT
    pltpu.sync_copy(z_hbm.at[pl.ds(r0, _RPT)], cnt_sh.at[pl.ds(r0, _RPT)])
    pltpu.sync_copy(ones_hbm, ones_v)
    plsc.subcore_barrier()

    def body(i, carry):
        pltpu.sync_copy(ones_v, cnt_sh.at[dst_v.at[i]], add=True)
        return carry

    lax.fori_loop(0, _NCHUNK, body, 0)
    plsc.subcore_barrier()

    pltpu.sync_copy(cnt_sh.at[pl.ds(r0, _RPT)],
                    cnt_out.at[c, pl.ds(r0, _RPT)])


_BR = 400


def _tc_layer(h, sum2, cnt2, WlT, WrT, bl2d):
    def body(h_ref, s_ref, c_ref, wl_ref, wr_ref, b_ref, o_ref):
        ssum = s_ref[0] + s_ref[1]
        cnt = c_ref[0][:, 0:1] + c_ref[1][:, 0:1]
        mean = ssum / jnp.maximum(cnt, 1.0)
        acc = jax.lax.dot(mean, wl_ref[...],
                          precision=jax.lax.Precision.HIGHEST,
                          preferred_element_type=jnp.float32)
        acc = acc + jax.lax.dot(h_ref[...], wr_ref[...],
                                precision=jax.lax.Precision.HIGHEST,
                                preferred_element_type=jnp.float32)
        o_ref[...] = jnp.maximum(acc + b_ref[...], 0.0)

    return pl.pallas_call(
        body,
        grid=(_N // _BR,),
        in_specs=[
            pl.BlockSpec((_BR, _D), lambda i: (i, 0)),
            pl.BlockSpec((_NC, _BR, _D), lambda i: (0, i, 0)),
            pl.BlockSpec((_NC, _BR, _D), lambda i: (0, i, 0)),
            pl.BlockSpec((_D, _D), lambda i: (0, 0)),
            pl.BlockSpec((_D, _D), lambda i: (0, 0)),
            pl.BlockSpec((1, _D), lambda i: (0, 0)),
        ],
        out_specs=pl.BlockSpec((_BR, _D), lambda i: (i, 0)),
        out_shape=jax.ShapeDtypeStruct((_N, _D), jnp.float32),
    )(h, sum2, cnt2, WlT, WrT, bl2d)


def kernel(x, edge_index, Wl1, bl1, Wr1, Wl2, bl2, Wr2, Wl3, bl3, Wr3):
    pad = _EPWP - _EPW
    srcw = edge_index[0].reshape(_NW, _EPW)
    dstw = edge_index[1].reshape(_NW, _EPW)
    src3 = jnp.pad(srcw, ((0, 0), (0, pad))).reshape(_NW, _NCHUNK, _CHUNK)
    dst3 = jnp.pad(dstw, ((0, 0), (0, pad)),
                   constant_values=_PADROW).reshape(_NW, _NCHUNK, _CHUNK)
    zeros = jnp.zeros((_NP, _D), jnp.float32)
    ones = jnp.ones((_CHUNK, _D), jnp.float32)

    h = x
    cnt2 = _sc_degree(dst3, zeros, ones)
    for Wl, bl, Wr in [(Wl1, bl1, Wr1), (Wl2, bl2, Wr2), (Wl3, bl3, Wr3)]:
        sum2 = _sc_segsum(h, edge_index[0], edge_index[1], zeros)
        h = _tc_layer(h, sum2[:, :_N], cnt2[:, :_N], Wl.T, Wr.T,
                      bl.reshape(1, _D))
    return h


# speedup vs baseline: 2.1827x; 1.0023x over previous
import functools

import jax
import jax.numpy as jnp
from jax import lax
from jax.experimental import pallas as pl
from jax.experimental.pallas import tpu as pltpu
from jax.experimental.pallas import tpu_sc as plsc

_N = 10000
_D = 128
_E = 320000

_NC = 2
_NS = 16
_NW = _NC * _NS
_EPW = _E // _NW
_CHUNK = 80
_NCHUNK = 128
_EPWP = _NCHUNK * _CHUNK
_NP = 10240
_RPT = _NP // _NS
_PADROW = _NP - _CHUNK

_mesh = plsc.VectorSubcoreMesh(core_axis_name="c", subcore_axis_name="s")


_SCH = 80
_SNCH = _EPW // _SCH


@functools.partial(
    pl.kernel,
    out_type=jax.ShapeDtypeStruct((_NC, _NP, _D), jnp.float32),
    mesh=_mesh,
    scratch_types=(
        [pltpu.VMEM((_SCH,), jnp.int32)] * 8
        + [pltpu.VMEM((_SCH, _D), jnp.float32)] * 4
        + [
            pltpu.VMEM_SHARED((_NP, _D), jnp.float32),
            pltpu.SemaphoreType.DMA,
            pltpu.SemaphoreType.DMA,
            pltpu.SemaphoreType.DMA,
            pltpu.SemaphoreType.DMA,
        ]
    ))
def _sc_segsum(h_hbm, src_hbm, dst_hbm, z_hbm, sum_out,
               srcA, dstA, srcB, dstB, srcC, dstC, srcD, dstD,
               rows0, rows1, rows2, rows3, acc_sh, sem0, sem1, sem2, sem3):
    c = lax.axis_index("c")
    s = lax.axis_index("s")
    wid = c * _NS + s

    def load_idx(i, src_v, dst_v):
        base = pl.multiple_of(wid * _EPW + i * _SCH, 8)
        pltpu.sync_copy(src_hbm.at[pl.ds(base, _SCH)], src_v)
        pltpu.sync_copy(dst_hbm.at[pl.ds(base, _SCH)], dst_v)

    r0 = s * _RPT
    pltpu.sync_copy(z_hbm.at[pl.ds(r0, _RPT)], acc_sh.at[pl.ds(r0, _RPT)])
    load_idx(0, srcA, dstA)
    plsc.subcore_barrier()

    def body(j, carry):
        i = 4 * j
        dg0 = pltpu.async_copy(h_hbm.at[srcA], rows0, sem0)
        load_idx(i + 1, srcB, dstB)
        dg1 = pltpu.async_copy(h_hbm.at[srcB], rows1, sem1)
        load_idx(i + 2, srcC, dstC)
        dg2 = pltpu.async_copy(h_hbm.at[srcC], rows2, sem2)
        load_idx(i + 3, srcD, dstD)
        dg3 = pltpu.async_copy(h_hbm.at[srcD], rows3, sem3)
        dg0.wait()
        pltpu.sync_copy(rows0, acc_sh.at[dstA], add=True)
        load_idx(i + 4, srcA, dstA)
        dg1.wait()
        pltpu.sync_copy(rows1, acc_sh.at[dstB], add=True)
        dg2.wait()
        pltpu.sync_copy(rows2, acc_sh.at[dstC], add=True)
        dg3.wait()
        pltpu.sync_copy(rows3, acc_sh.at[dstD], add=True)
        return carry

    lax.fori_loop(0, _SNCH // 4, body, 0)
    pltpu.async_copy(h_hbm.at[srcA], rows0, sem0).wait()
    pltpu.sync_copy(rows0, acc_sh.at[dstA], add=True)
    plsc.subcore_barrier()

    pltpu.sync_copy(acc_sh.at[pl.ds(r0, _RPT)],
                    sum_out.at[c, pl.ds(r0, _RPT)])


@functools.partial(
    pl.kernel,
    out_type=jax.ShapeDtypeStruct((_NC, _NP, _D), jnp.float32),
    mesh=_mesh,
    scratch_types=[
        pltpu.VMEM((_NCHUNK, _CHUNK), jnp.int32),
        pltpu.VMEM((_CHUNK, _D), jnp.float32),
        pltpu.VMEM_SHARED((_NP, _D), jnp.float32),
        pltpu.SemaphoreType.DMA,
        pltpu.SemaphoreType.DMA,
        pltpu.SemaphoreType.DMA,
        pltpu.SemaphoreType.DMA,
    ])
def _sc_degree(dst_hbm, z_hbm, ones_hbm, cnt_out, dst_v, ones_v, cnt_sh,
               sem0, sem1, sem2, sem3):
    c = lax.axis_index("c")
    s = lax.axis_index("s")

    pltpu.sync_copy(dst_hbm.at[c * _NS + s], dst_v)
    r0 = s * _RPT
    pltpu.sync_copy(z_hbm.at[pl.ds(r0, _RPT)], cnt_sh.at[pl.ds(r0, _RPT)])
    pltpu.sync_copy(ones_hbm, ones_v)
    plsc.subcore_barrier()

    def body(j, carry):
        i = 4 * j
        d0 = pltpu.async_copy(ones_v, cnt_sh.at[dst_v.at[i]], sem0, add=True)
        d1 = pltpu.async_copy(ones_v, cnt_sh.at[dst_v.at[i + 1]], sem1,
                              add=True)
        d2 = pltpu.async_copy(ones_v, cnt_sh.at[dst_v.at[i + 2]], sem2,
                              add=True)
        d3 = pltpu.async_copy(ones_v, cnt_sh.at[dst_v.at[i + 3]], sem3,
                              add=True)
        d0.wait()
        d1.wait()
        d2.wait()
        d3.wait()
        return carry

    lax.fori_loop(0, _NCHUNK // 4, body, 0)
    plsc.subcore_barrier()

    pltpu.sync_copy(cnt_sh.at[pl.ds(r0, _RPT)],
                    cnt_out.at[c, pl.ds(r0, _RPT)])


_BR = 400


def _tc_layer(h, sum2, cnt2, WlT, WrT, bl2d):
    def body(h_ref, s_ref, c_ref, wl_ref, wr_ref, b_ref, o_ref):
        ssum = s_ref[0] + s_ref[1]
        cnt = c_ref[0][:, 0:1] + c_ref[1][:, 0:1]
        mean = ssum / jnp.maximum(cnt, 1.0)
        acc = jax.lax.dot(mean, wl_ref[...],
                          precision=jax.lax.Precision.HIGHEST,
                          preferred_element_type=jnp.float32)
        acc = acc + jax.lax.dot(h_ref[...], wr_ref[...],
                                precision=jax.lax.Precision.HIGHEST,
                                preferred_element_type=jnp.float32)
        o_ref[...] = jnp.maximum(acc + b_ref[...], 0.0)

    return pl.pallas_call(
        body,
        grid=(_N // _BR,),
        in_specs=[
            pl.BlockSpec((_BR, _D), lambda i: (i, 0)),
            pl.BlockSpec((_NC, _BR, _D), lambda i: (0, i, 0)),
            pl.BlockSpec((_NC, _BR, _D), lambda i: (0, i, 0)),
            pl.BlockSpec((_D, _D), lambda i: (0, 0)),
            pl.BlockSpec((_D, _D), lambda i: (0, 0)),
            pl.BlockSpec((1, _D), lambda i: (0, 0)),
        ],
        out_specs=pl.BlockSpec((_BR, _D), lambda i: (i, 0)),
        out_shape=jax.ShapeDtypeStruct((_N, _D), jnp.float32),
    )(h, sum2, cnt2, WlT, WrT, bl2d)


def kernel(x, edge_index, Wl1, bl1, Wr1, Wl2, bl2, Wr2, Wl3, bl3, Wr3):
    pad = _EPWP - _EPW
    srcw = edge_index[0].reshape(_NW, _EPW)
    dstw = edge_index[1].reshape(_NW, _EPW)
    src3 = jnp.pad(srcw, ((0, 0), (0, pad))).reshape(_NW, _NCHUNK, _CHUNK)
    dst3 = jnp.pad(dstw, ((0, 0), (0, pad)),
                   constant_values=_PADROW).reshape(_NW, _NCHUNK, _CHUNK)
    zeros = jnp.zeros((_NP, _D), jnp.float32)
    ones = jnp.ones((_CHUNK, _D), jnp.float32)

    h = x
    cnt2 = _sc_degree(dst3, zeros, ones)
    for Wl, bl, Wr in [(Wl1, bl1, Wr1), (Wl2, bl2, Wr2), (Wl3, bl3, Wr3)]:
        sum2 = _sc_segsum(h, edge_index[0], edge_index[1], zeros)
        h = _tc_layer(h, sum2[:, :_N], cnt2[:, :_N], Wl.T, Wr.T,
                      bl.reshape(1, _D))
    return h


# speedup vs baseline: 2.1828x; 1.0001x over previous
import functools

import jax
import jax.numpy as jnp
from jax import lax
from jax.experimental import pallas as pl
from jax.experimental.pallas import tpu as pltpu
from jax.experimental.pallas import tpu_sc as plsc

_N = 10000
_D = 128
_E = 320000

_NC = 2
_NS = 16
_NW = _NC * _NS
_EPW = _E // _NW
_CHUNK = 80
_NCHUNK = 128
_EPWP = _NCHUNK * _CHUNK
_NP = 10240
_RPT = _NP // _NS
_PADROW = _NP - _CHUNK

_mesh = plsc.VectorSubcoreMesh(core_axis_name="c", subcore_axis_name="s")


_SCH = 80
_SNCH = _EPW // _SCH


@functools.partial(
    pl.kernel,
    out_type=jax.ShapeDtypeStruct((_NC, _NP, _D), jnp.float32),
    mesh=_mesh,
    scratch_types=(
        [pltpu.VMEM((_SCH,), jnp.int32)] * 8
        + [pltpu.VMEM((_SCH, _D), jnp.float32)] * 4
        + [
            pltpu.VMEM_SHARED((_NP, _D), jnp.float32),
            pltpu.SemaphoreType.DMA,
            pltpu.SemaphoreType.DMA,
            pltpu.SemaphoreType.DMA,
            pltpu.SemaphoreType.DMA,
        ]
    ))
def _sc_segsum(h_hbm, src_hbm, dst_hbm, z_hbm, sum_out,
               srcA, dstA, srcB, dstB, srcC, dstC, srcD, dstD,
               rows0, rows1, rows2, rows3, acc_sh, sem0, sem1, sem2, sem3):
    c = lax.axis_index("c")
    s = lax.axis_index("s")
    wid = c * _NS + s

    def load_idx(i, src_v, dst_v):
        base = pl.multiple_of(wid * _EPW + i * _SCH, 8)
        pltpu.sync_copy(src_hbm.at[pl.ds(base, _SCH)], src_v)
        pltpu.sync_copy(dst_hbm.at[pl.ds(base, _SCH)], dst_v)

    r0 = s * _RPT
    pltpu.sync_copy(z_hbm.at[pl.ds(r0, _RPT)], acc_sh.at[pl.ds(r0, _RPT)])
    load_idx(0, srcA, dstA)
    plsc.subcore_barrier()

    def body(j, carry):
        i = 4 * j
        dg0 = pltpu.async_copy(h_hbm.at[srcA], rows0, sem0)
        load_idx(i + 1, srcB, dstB)
        dg1 = pltpu.async_copy(h_hbm.at[srcB], rows1, sem1)
        load_idx(i + 2, srcC, dstC)
        dg2 = pltpu.async_copy(h_hbm.at[srcC], rows2, sem2)
        load_idx(i + 3, srcD, dstD)
        dg3 = pltpu.async_copy(h_hbm.at[srcD], rows3, sem3)
        dg0.wait()
        pltpu.sync_copy(rows0, acc_sh.at[dstA], add=True)
        load_idx(i + 4, srcA, dstA)
        dg1.wait()
        pltpu.sync_copy(rows1, acc_sh.at[dstB], add=True)
        dg2.wait()
        pltpu.sync_copy(rows2, acc_sh.at[dstC], add=True)
        dg3.wait()
        pltpu.sync_copy(rows3, acc_sh.at[dstD], add=True)
        return carry

    lax.fori_loop(0, _SNCH // 4, body, 0)
    if _SNCH % 4:
        pltpu.async_copy(h_hbm.at[srcA], rows0, sem0).wait()
        pltpu.sync_copy(rows0, acc_sh.at[dstA], add=True)
    plsc.subcore_barrier()

    pltpu.sync_copy(acc_sh.at[pl.ds(r0, _RPT)],
                    sum_out.at[c, pl.ds(r0, _RPT)])


@functools.partial(
    pl.kernel,
    out_type=jax.ShapeDtypeStruct((_NC, _NP, _D), jnp.float32),
    mesh=_mesh,
    scratch_types=[
        pltpu.VMEM((_NCHUNK, _CHUNK), jnp.int32),
        pltpu.VMEM((_CHUNK, _D), jnp.float32),
        pltpu.VMEM_SHARED((_NP, _D), jnp.float32),
        pltpu.SemaphoreType.DMA,
        pltpu.SemaphoreType.DMA,
        pltpu.SemaphoreType.DMA,
        pltpu.SemaphoreType.DMA,
    ])
def _sc_degree(dst_hbm, z_hbm, ones_hbm, cnt_out, dst_v, ones_v, cnt_sh,
               sem0, sem1, sem2, sem3):
    c = lax.axis_index("c")
    s = lax.axis_index("s")

    pltpu.sync_copy(dst_hbm.at[c * _NS + s], dst_v)
    r0 = s * _RPT
    pltpu.sync_copy(z_hbm.at[pl.ds(r0, _RPT)], cnt_sh.at[pl.ds(r0, _RPT)])
    pltpu.sync_copy(ones_hbm, ones_v)
    plsc.subcore_barrier()

    def body(j, carry):
        i = 4 * j
        d0 = pltpu.async_copy(ones_v, cnt_sh.at[dst_v.at[i]], sem0, add=True)
        d1 = pltpu.async_copy(ones_v, cnt_sh.at[dst_v.at[i + 1]], sem1,
                              add=True)
        d2 = pltpu.async_copy(ones_v, cnt_sh.at[dst_v.at[i + 2]], sem2,
                              add=True)
        d3 = pltpu.async_copy(ones_v, cnt_sh.at[dst_v.at[i + 3]], sem3,
                              add=True)
        d0.wait()
        d1.wait()
        d2.wait()
        d3.wait()
        return carry

    lax.fori_loop(0, _NCHUNK // 4, body, 0)
    plsc.subcore_barrier()

    pltpu.sync_copy(cnt_sh.at[pl.ds(r0, _RPT)],
                    cnt_out.at[c, pl.ds(r0, _RPT)])


_BR = 400


def _tc_layer(h, sum2, cnt2, WlT, WrT, bl2d):
    def body(h_ref, s_ref, c_ref, wl_ref, wr_ref, b_ref, o_ref):
        ssum = s_ref[0] + s_ref[1]
        cnt = c_ref[0][:, 0:1] + c_ref[1][:, 0:1]
        mean = ssum / jnp.maximum(cnt, 1.0)
        acc = jax.lax.dot(mean, wl_ref[...],
                          precision=jax.lax.Precision.HIGHEST,
                          preferred_element_type=jnp.float32)
        acc = acc + jax.lax.dot(h_ref[...], wr_ref[...],
                                precision=jax.lax.Precision.HIGHEST,
                                preferred_element_type=jnp.float32)
        o_ref[...] = jnp.maximum(acc + b_ref[...], 0.0)

    return pl.pallas_call(
        body,
        grid=(_N // _BR,),
        in_specs=[
            pl.BlockSpec((_BR, _D), lambda i: (i, 0)),
            pl.BlockSpec((_NC, _BR, _D), lambda i: (0, i, 0)),
            pl.BlockSpec((_NC, _BR, _D), lambda i: (0, i, 0)),
            pl.BlockSpec((_D, _D), lambda i: (0, 0)),
            pl.BlockSpec((_D, _D), lambda i: (0, 0)),
            pl.BlockSpec((1, _D), lambda i: (0, 0)),
        ],
        out_specs=pl.BlockSpec((_BR, _D), lambda i: (i, 0)),
        out_shape=jax.ShapeDtypeStruct((_N, _D), jnp.float32),
    )(h, sum2, cnt2, WlT, WrT, bl2d)


def kernel(x, edge_index, Wl1, bl1, Wr1, Wl2, bl2, Wr2, Wl3, bl3, Wr3):
    pad = _EPWP - _EPW
    srcw = edge_index[0].reshape(_NW, _EPW)
    dstw = edge_index[1].reshape(_NW, _EPW)
    src3 = jnp.pad(srcw, ((0, 0), (0, pad))).reshape(_NW, _NCHUNK, _CHUNK)
    dst3 = jnp.pad(dstw, ((0, 0), (0, pad)),
                   constant_values=_PADROW).reshape(_NW, _NCHUNK, _CHUNK)
    zeros = jnp.zeros((_NP, _D), jnp.float32)
    ones = jnp.ones((_CHUNK, _D), jnp.float32)

    h = x
    cnt2 = _sc_degree(dst3, zeros, ones)
    for Wl, bl, Wr in [(Wl1, bl1, Wr1), (Wl2, bl2, Wr2), (Wl3, bl3, Wr3)]:
        sum2 = _sc_segsum(h, edge_index[0], edge_index[1], zeros)
        h = _tc_layer(h, sum2[:, :_N], cnt2[:, :_N], Wl.T, Wr.T,
                      bl.reshape(1, _D))
    return h
